# Initial kernel scaffold; baseline (speedup 1.0000x reference)
import jax
import jax.numpy as jnp
from jax.experimental import pallas as pl


def kernel(news_input, click_history, outputs_table, neighbors, Wq, bq, Wk, bk, Wv, bv, ln1_g, ln1_b, pW, pb, pq, ln2_g, ln2_b):
    raise NotImplementedError("write your pallas kernel here")


# speedup vs baseline: 1.9435x; 1.9435x over previous
import functools
import math

import jax
import jax.numpy as jnp
from jax import lax
from jax.experimental import pallas as pl
from jax.experimental.pallas import tpu as pltpu
from jax.experimental.pallas import tpu_sc as plsc

B, N, D = 16, 50, 400
K, G = 10000, 20
HN, HD = 20, 20
AH = 200
STEPS = 6
SLOTS = 10

_GDN = lax.GatherDimensionNumbers(offset_dims=(), collapsed_slice_dims=(0,),
                                  start_index_map=(0,))


def _lane_perm(v, idx16):
    return lax.gather(v, idx16[:, None], _GDN, slice_sizes=(1,),
                      mode=lax.GatherScatterMode.PROMISE_IN_BOUNDS)


def _vsum16(v):
    lanes = lax.broadcasted_iota(jnp.int32, (16,), 0)
    for sh in (8, 4, 2, 1):
        v = v + _lane_perm(v, lanes ^ sh)
    return v[0]


NW = 32
IPW = (B * N) // NW
HB = 160
SEQ = 2 * HB
REAL = STEPS * IPW
LN200 = math.log(200.0)


def _walk_body(news_hbm, click_hbm, table_hbm, neigh_hbm, out_hbm,
               news_v, click_v, group_v, neigh_v, obuf):
    f32 = jnp.float32
    i32 = jnp.int32
    wid = lax.axis_index("s") * 2 + lax.axis_index("c")
    base = wid * IPW
    zeros16 = jnp.zeros((16,), f32)

    def _zr(r, _):
        def _zc(j, _):
            obuf[pl.ds(r * D + 16 * j, 16)] = zeros16
            return 0
        return lax.fori_loop(0, D // 16, _zc, 0)
    lax.fori_loop(REAL, HB, _zr, 0)

    pltpu.sync_copy(news_hbm.at[pl.ds(base * D, IPW * D)], news_v)
    pltpu.sync_copy(click_hbm.at[pl.ds(wid * 32, 32)], click_v.at[pl.ds(0, 32)])

    def item_body(i, _):
        idx_init = click_v[pl.ds(i, 16)][0]

        def step_body(t, carry):
            idx0, mode0, sel0 = carry

            def walk_fn(car):
                idx, _, _ = car
                pltpu.sync_copy(table_hbm.at[idx - 1], group_v)
                pltpu.sync_copy(neigh_hbm.at[pl.ds(idx * 32, 32)],
                                neigh_v.at[pl.ds(0, 32)])

                accs = [None] * G
                for j in range(D // 16):
                    nj = news_v[pl.ds(i * D + 16 * j, 16)]
                    for g in range(G):
                        p = group_v[g, pl.ds(16 * j, 16)] * nj
                        accs[g] = p if j == 0 else accs[g] + p

                ninf = f32(-3.4e38)
                mx = ninf
                mi = i32(0)
                nmx = ninf
                nmi = i32(0)
                hnz = i32(0)
                for g in range(G):
                    s = _vsum16(accs[g])
                    upd = s > mx
                    mi = jnp.where(upd, i32(g), mi)
                    mx = jnp.where(upd, s, mx)
                    nz = s != 0.0
                    nupd = jnp.logical_and(nz, s > nmx)
                    nmi = jnp.where(nupd, i32(g), nmi)
                    nmx = jnp.where(nupd, s, nmx)
                    hnz = jnp.where(nz, 1, hnz)

                hit_zero = mx == 0.0
                fallback = jnp.logical_and(hit_zero, hnz > 0)
                go_zero = jnp.logical_and(hit_zero, hnz == 0)
                sel = jnp.where(fallback, nmi, mi)
                hop = neigh_v[pl.ds(nmi, 16)][0]
                new_idx = jnp.where(fallback, hop, idx)
                new_mode = jnp.where(go_zero, i32(2),
                                     jnp.where(fallback, i32(0), i32(1)))
                return new_idx, new_mode, sel

            idx1, mode1, sel1 = lax.cond(mode0 == 0, walk_fn,
                                         lambda car: car,
                                         (idx0, mode0, sel0))
            row = t * IPW + i

            def w_body(j, _):
                val = group_v[sel1, pl.ds(16 * j, 16)]
                obuf[pl.ds(row * D + 16 * j, 16)] = jnp.where(mode1 == 2,
                                                              zeros16, val)
                return 0
            lax.fori_loop(0, D // 16, w_body, 0)
            return idx1, mode1, sel1

        lax.fori_loop(0, STEPS, step_body,
                      (idx_init, jnp.int32(0), jnp.int32(0)))
        return 0

    lax.fori_loop(0, IPW, item_body, 0)
    pltpu.sync_copy(obuf, out_hbm.at[pl.ds(wid * HB * D, HB * D)])


def _walk_sc(news_flat, click_flat, outputs_table, neigh_flat):
    mesh = plsc.VectorSubcoreMesh(core_axis_name="c", subcore_axis_name="s")
    run = functools.partial(
        pl.kernel,
        out_type=jax.ShapeDtypeStruct((NW * HB * D,), jnp.float32),
        mesh=mesh,
        scratch_types=[
            pltpu.VMEM((IPW * D,), jnp.float32),
            pltpu.VMEM((48,), jnp.int32),
            pltpu.VMEM((G, D), jnp.float32),
            pltpu.VMEM((48,), jnp.int32),
            pltpu.VMEM((HB * D,), jnp.float32),
        ],
    )(_walk_body)
    return run(news_flat, click_flat, outputs_table, neigh_flat)


def _att_body(x_ref, wq_ref, bq_ref, wk_ref, bk_ref, wv_ref, bv_ref,
              g1_ref, b1_ref, pw_ref, pb_ref, pq_ref, g2_ref, b2_ref,
              o_ref):
    f32 = jnp.float32
    x = x_ref[0]
    q = jnp.dot(x, wq_ref[...], preferred_element_type=f32) + bq_ref[...]
    k = jnp.dot(x, wk_ref[...], preferred_element_type=f32) + bk_ref[...]
    v = jnp.dot(x, wv_ref[...], preferred_element_type=f32) + bv_ref[...]
    riota = lax.broadcasted_iota(jnp.int32, (1, SEQ), 1)
    rmod = riota % HB
    kbias = jnp.where(rmod > REAL, f32(-1e30), f32(0.0))
    kbias = jnp.where(riota == REAL, f32(LN200), kbias)
    kbias = jnp.where(riota == HB + REAL, f32(-1e30), kbias)
    scale = f32(1.0 / math.sqrt(HD))
    outs = []
    for h in range(HN):
        qh = q[:, HD * h:HD * (h + 1)]
        kh = k[:, HD * h:HD * (h + 1)]
        vh = v[:, HD * h:HD * (h + 1)]
        sc = lax.dot_general(qh, kh, (((1,), (1,)), ((), ())),
                             preferred_element_type=f32) * scale + kbias
        m = jnp.max(sc, axis=1, keepdims=True)
        e = jnp.exp(sc - m)
        att = e / jnp.sum(e, axis=1, keepdims=True)
        outs.append(lax.dot_general(att, vh, (((1,), (0,)), ((), ())),
                                    preferred_element_type=f32))
    y = jnp.concatenate(outs, axis=1)
    mu = jnp.mean(y, axis=1, keepdims=True)
    var = jnp.mean((y - mu) ** 2, axis=1, keepdims=True)
    yn = (y - mu) / jnp.sqrt(var + 1e-5) * g1_ref[...] + b1_ref[...]
    t = jnp.tanh(jnp.dot(yn, pw_ref[...], preferred_element_type=f32)
                 + pb_ref[...])
    e_all = jnp.sum(t * pq_ref[...], axis=1, keepdims=True)
    es = [jnp.concatenate([e_all[IPW * s0:IPW * (s0 + 1)],
                           e_all[HB + IPW * s0:HB + IPW * (s0 + 1)]],
                          axis=0)
          for s0 in range(STEPS)]
    epad = jnp.broadcast_to(e_all[REAL:REAL + 1], (N, SLOTS - STEPS))
    ee = jnp.concatenate(es + [epad], axis=1)
    em = jnp.max(ee, axis=1, keepdims=True)
    ex = jnp.exp(ee - em)
    a = ex / jnp.sum(ex, axis=1, keepdims=True)

    def slot_rows(s0):
        return jnp.concatenate([yn[IPW * s0:IPW * (s0 + 1)],
                                yn[HB + IPW * s0:HB + IPW * (s0 + 1)]],
                               axis=0)

    acc = a[:, 0:1] * slot_rows(0)
    for s0 in range(1, STEPS):
        acc = acc + a[:, s0:s0 + 1] * slot_rows(s0)
    apad = jnp.sum(a[:, STEPS:SLOTS], axis=1, keepdims=True)
    acc = acc + apad * yn[REAL:REAL + 1]
    mu2 = jnp.mean(acc, axis=1, keepdims=True)
    var2 = jnp.mean((acc - mu2) ** 2, axis=1, keepdims=True)
    o_ref[0] = ((acc - mu2) / jnp.sqrt(var2 + 1e-5) * g2_ref[...]
                + b2_ref[...])


def _att_tc(x, Wq, bq, Wk, bk, Wv, bv, ln1_g, ln1_b, pW, pb, pq,
            ln2_g, ln2_b):
    f32 = jnp.float32

    def im0(i):
        return (0, 0)

    w_spec = pl.BlockSpec((D, D), im0)
    b_spec = pl.BlockSpec((1, D), im0)
    pw_spec = pl.BlockSpec((D, AH), im0)
    pb_spec = pl.BlockSpec((1, AH), im0)
    return pl.pallas_call(
        _att_body,
        grid=(B,),
        in_specs=[pl.BlockSpec((1, SEQ, D), lambda i: (i, 0, 0)),
                  w_spec, b_spec, w_spec, b_spec, w_spec, b_spec,
                  b_spec, b_spec, pw_spec, pb_spec, pb_spec,
                  b_spec, b_spec],
        out_specs=pl.BlockSpec((1, N, D), lambda i: (i, 0, 0)),
        out_shape=jax.ShapeDtypeStruct((B, N, D), f32),
    )(x, Wq, bq.reshape(1, D), Wk, bk.reshape(1, D), Wv, bv.reshape(1, D),
      ln1_g.reshape(1, D), ln1_b.reshape(1, D), pW, pb.reshape(1, AH),
      pq.reshape(1, AH), ln2_g.reshape(1, D), ln2_b.reshape(1, D))


def kernel(news_input, click_history, outputs_table, neighbors,
           Wq, bq, Wk, bk, Wv, bv, ln1_g, ln1_b, pW, pb, pq, ln2_g, ln2_b):
    news_flat = news_input.reshape(B * N * D)
    click_flat = jnp.pad(click_history.reshape(NW, IPW),
                         ((0, 0), (0, 32 - IPW))).reshape(NW * 32)
    neigh_flat = jnp.pad(neighbors, ((0, 0), (0, 32 - G))).reshape(-1)
    xw = _walk_sc(news_flat, click_flat, outputs_table, neigh_flat)
    x = xw.reshape(B, SEQ, D)
    return _att_tc(x, Wq, bq, Wk, bk, Wv, bv, ln1_g, ln1_b,
                   pW, pb, pq, ln2_g, ln2_b)
